# baseline (speedup 1.0000x reference)
import jax
import jax.numpy as jnp
from jax import lax
from jax.experimental import pallas as pl
from jax.experimental.pallas import tpu as pltpu
from jax.experimental.pallas import tpu_sc as plsc

N_FIELDS = 26
BATCH = 16384
VOCAB = 100000
EMBED_DIM = 64

NUM_CORES = 2
NUM_SUBCORES = 16
NUM_WORKERS = NUM_CORES * NUM_SUBCORES
D_PER_W = EMBED_DIM // NUM_WORKERS

NQ = 4
Q_STARTS = (0, 24960, 49920, 74880)
Q_LENS = (25088, 25088, 25088, 25120)
SBUF = 25120
SLOT = 17408
LBUF = 21504
CHK = 1024
IDXC = 2048
DUMP = BATCH


def _sc_body(t_hbm, feats_hbm, out_hbm, lists_hbm, cnts_hbm,
             sbuf_a, sbuf_b, lbuf_a, lbuf_b, acc_v, idx_v, cnt_v, cnt26_v,
             sem0, sem1, lsem):
    c = lax.axis_index("c")
    s = lax.axis_index("s")
    w = c * NUM_SUBCORES + s

    iota16 = lax.iota(jnp.int32, 16)
    build = [sbuf_a, sbuf_b, lbuf_a, lbuf_b]

    def do_field(fv):
        def chunk_body(k, tails):
            pltpu.sync_copy(
                feats_hbm.at[pl.ds(fv * BATCH + k * IDXC, IDXC)], idx_v)

            def grp_body(j, tails):
                iv = idx_v[pl.ds(j * 16, 16)]
                ph = ((k * IDXC + j * 16) + iota16) << 16
                new_tails = []
                for t in range(NQ):
                    lo = Q_STARTS[t]
                    hi = Q_STARTS[t + 1] if t + 1 < NQ else VOCAB
                    m = (iv >= lo) & (iv < hi) if t > 0 else (iv < hi)
                    plsc.store_compressed(
                        build[t].at[pl.ds(tails[t], 16)],
                        plsc.bitcast((iv - lo) | ph, jnp.float32), mask=m)
                    pc = jnp.max(plsc.all_reduce_population_count(m))
                    new_tails.append(tails[t] + pc)
                return tuple(new_tails)

            return lax.fori_loop(0, IDXC // 16, grp_body, tails)

        z = jnp.int32(0)
        tails = lax.fori_loop(0, BATCH // IDXC, chunk_body, (z, z, z, z))

        pad = plsc.bitcast((DUMP + iota16) << 16, jnp.float32)
        cvec = jnp.zeros((16,), jnp.int32)
        for t in range(NQ):
            build[t][pl.ds(tails[t], 16)] = pad
            cvec = jnp.where(iota16 == t, tails[t], cvec)
        cnt_v[pl.ds(0, 16)] = cvec
        pltpu.sync_copy(cnt_v, cnts_hbm.at[c, fv])

        for t in range(NQ):
            def cc_body(cc, _):
                pltpu.sync_copy(
                    build[t].at[pl.ds(cc * CHK, CHK)],
                    lists_hbm.at[c, fv, t, pl.ds(cc * CHK, CHK)])
                return 0
            nc = (tails[t] + 16 + CHK - 1) // CHK
            lax.fori_loop(0, nc, cc_body, 0)

    for rep in range(2):
        fv = s + rep * NUM_SUBCORES

        @pl.when(fv < N_FIELDS)
        def _():
            do_field(fv)

    plsc.subcore_barrier()

    pltpu.sync_copy(cnts_hbm.at[c], cnt26_v)

    def counts_of(f):
        row = cnt26_v[f, pl.ds(0, 16)]
        return [row[t] for t in range(NQ)]

    def offs_of(cnts):
        offs = [jnp.int32(0)]
        for t in range(NQ - 1):
            offs.append(offs[t] + ((cnts[t] + 16 + CHK - 1) // CHK) * CHK)
        return offs

    def list_xfer(f, buf, fire):
        cnts = counts_of(f)
        offs = offs_of(cnts)
        lb = lbuf_a if buf == 0 else lbuf_b
        for t in range(NQ):
            def cc_body(cc, _):
                cp = pltpu.make_async_copy(
                    lists_hbm.at[c, f, t, pl.ds(cc * CHK, CHK)],
                    lb.at[pl.ds(offs[t] + cc * CHK, CHK)], lsem)
                if fire:
                    cp.start()
                else:
                    cp.wait()
                return 0
            nc = (cnts[t] + 16 + CHK - 1) // CHK
            lax.fori_loop(0, nc, cc_body, 0)

    def slice_cp(f, d, t):
        sb = sbuf_a if t % 2 == 0 else sbuf_b
        return pltpu.make_async_copy(
            t_hbm.at[f, d, pl.ds(Q_STARTS[t], Q_LENS[t])],
            sb.at[pl.ds(0, Q_LENS[t])],
            sem0 if t % 2 == 0 else sem1)

    zeros = jnp.zeros((16,), jnp.float32)

    def dim_body(dl, _):
        d = w * D_PER_W + dl

        @plsc.parallel_loop(0, (BATCH + 16) // 16, unroll=8)
        def zero_body(j):
            acc_v[pl.ds(j * 16, 16)] = zeros

        list_xfer(0, 0, True)
        slice_cp(0, d, 0).start()
        slice_cp(0, d, 1).start()
        list_xfer(0, 0, False)

        def pair_body(fp, _):
            for fe in range(2):
                f = fp * 2 + fe
                lb = lbuf_a if fe == 0 else lbuf_b

                @pl.when(f < N_FIELDS - 1)
                def _():
                    list_xfer(f + 1, 1 - fe, True)

                cnts = counts_of(f)
                offs = offs_of(cnts)
                for t in range(NQ):
                    sb = sbuf_a if t % 2 == 0 else sbuf_b
                    slice_cp(f, d, t).wait()
                    base = offs[t]
                    iters = (cnts[t] >> 4) + 1

                    @plsc.parallel_loop(0, iters, unroll=8)
                    def gather_body(j):
                        e = plsc.bitcast(lb[pl.ds(base + j * 16, 16)],
                                         jnp.int32)
                        x = e & jnp.int32(0xFFFF)
                        p = lax.shift_right_logical(e, 16)
                        g = plsc.load_gather(sb, [x])
                        plsc.addupdate_scatter(acc_v, [p], g)

                    t2s = (t + 2) % NQ
                    fnext = f + (t + 2) // NQ

                    @pl.when(fnext < N_FIELDS)
                    def _():
                        slice_cp(fnext, d, t2s).start()

                @pl.when(f < N_FIELDS - 1)
                def _():
                    list_xfer(f + 1, 1 - fe, False)

            return 0

        lax.fori_loop(0, N_FIELDS // 2, pair_body, 0)
        pltpu.sync_copy(acc_v.at[pl.ds(0, BATCH)], out_hbm.at[d])
        return 0

    lax.fori_loop(0, D_PER_W, dim_body, 0)


@jax.jit
def _embed_sum(t_tr, feats_flat):
    mesh = plsc.VectorSubcoreMesh(core_axis_name="c", subcore_axis_name="s")
    kfn = pl.kernel(
        _sc_body,
        out_type=(
            jax.ShapeDtypeStruct((EMBED_DIM, BATCH), jnp.float32),
            jax.ShapeDtypeStruct((NUM_CORES, N_FIELDS, NQ, SLOT),
                                 jnp.float32),
            jax.ShapeDtypeStruct((NUM_CORES, N_FIELDS, 16), jnp.int32),
        ),
        mesh=mesh,
        scratch_types=[
            pltpu.VMEM((SBUF,), jnp.float32),
            pltpu.VMEM((SBUF,), jnp.float32),
            pltpu.VMEM((LBUF,), jnp.float32),
            pltpu.VMEM((LBUF,), jnp.float32),
            pltpu.VMEM((BATCH + 16,), jnp.float32),
            pltpu.VMEM((IDXC,), jnp.int32),
            pltpu.VMEM((16,), jnp.int32),
            pltpu.VMEM((N_FIELDS, 16), jnp.int32),
            pltpu.SemaphoreType.DMA,
            pltpu.SemaphoreType.DMA,
            pltpu.SemaphoreType.DMA,
        ],
        compiler_params=pltpu.CompilerParams(use_tc_tiling_on_sc=True,
                                             needs_layout_passes=False),
    )
    return kfn(t_tr, feats_flat)


def kernel(features, tables):
    t_tr = tables.transpose(0, 2, 1)
    feats_flat = features.reshape(N_FIELDS * BATCH)
    out_t, _, _ = _embed_sum(t_tr, feats_flat)
    return out_t.T


# speedup vs baseline: 1.1135x; 1.1135x over previous
import jax
import jax.numpy as jnp
from jax import lax
from jax.experimental import pallas as pl
from jax.experimental.pallas import tpu as pltpu
from jax.experimental.pallas import tpu_sc as plsc

N_FIELDS = 26
BATCH = 16384
VOCAB = 100000
EMBED_DIM = 64

NUM_CORES = 2
NUM_SUBCORES = 16
NUM_WORKERS = NUM_CORES * NUM_SUBCORES
D_PER_W = EMBED_DIM // NUM_WORKERS

NQ = 6
Q_STARTS = (0, 16640, 33280, 49920, 66560, 83200)
Q_LENS = (16768, 16768, 16768, 16768, 16768, 16800)
SBUF = 16768
SBUFD = 16800
SLOT = 16384
LBUF = 20480
CHK = 512
IDXC = 2048
DUMP = BATCH


def _ncf(cnt):
    return (((cnt + 15) >> 4 << 4) + CHK - 1) // CHK


def _sc_body(t_hbm, feats_hbm, out_hbm, lists_hbm, cnts_hbm,
             sbuf_a, sbuf_b, sbuf_c, sbuf_d, lbuf_a, lbuf_b,
             acc_v, idx_v, cnt_v, cnt26_v, sem0, sem1, sem2, sem3, lsem):
    c = lax.axis_index("c")
    s = lax.axis_index("s")
    w = c * NUM_SUBCORES + s

    iota16 = lax.iota(jnp.int32, 16)
    build = [sbuf_a, sbuf_b, sbuf_c, sbuf_d, lbuf_a, lbuf_b]
    SB_OF = (sbuf_a, sbuf_b, sbuf_c, sbuf_a, sbuf_b, sbuf_d)
    SEM_OF = (sem0, sem1, sem2, sem0, sem1, sem3)
    NEXT_OF = ((0, 3), (0, 4), (1, 2), (1, 0), (1, 1), (1, 5))

    def do_field(fv):
        def chunk_body(k, tails):
            pltpu.sync_copy(
                feats_hbm.at[pl.ds(fv * BATCH + k * IDXC, IDXC)], idx_v)

            def grp_body(j, tails):
                iv = idx_v[pl.ds(j * 16, 16)]
                ph = ((k * IDXC + j * 16) + iota16) << 16
                new_tails = []
                for t in range(NQ):
                    lo = Q_STARTS[t]
                    hi = Q_STARTS[t + 1] if t + 1 < NQ else VOCAB
                    m = (iv >= lo) & (iv < hi) if t > 0 else (iv < hi)
                    plsc.store_compressed(
                        build[t].at[pl.ds(tails[t], 16)],
                        plsc.bitcast((iv - lo) | ph, jnp.float32), mask=m)
                    pc = jnp.max(plsc.all_reduce_population_count(m))
                    new_tails.append(tails[t] + pc)
                return tuple(new_tails)

            return lax.fori_loop(0, IDXC // 16, grp_body, tails)

        z = jnp.int32(0)
        tails = lax.fori_loop(0, BATCH // IDXC, chunk_body,
                              (z, z, z, z, z, z))

        pad = plsc.bitcast((DUMP + iota16) << 16, jnp.float32)
        cvec = jnp.zeros((16,), jnp.int32)
        for t in range(NQ):
            build[t][pl.ds(tails[t], 16)] = pad
            cvec = jnp.where(iota16 == t, tails[t], cvec)
        cnt_v[pl.ds(0, 16)] = cvec
        pltpu.sync_copy(cnt_v, cnts_hbm.at[c, fv])

        for t in range(NQ):
            def cc_body(cc, _):
                pltpu.sync_copy(
                    build[t].at[pl.ds(cc * CHK, CHK)],
                    lists_hbm.at[c, fv, t, pl.ds(cc * CHK, CHK)])
                return 0
            lax.fori_loop(0, _ncf(tails[t]), cc_body, 0)

    for rep in range(2):
        fv = s + rep * NUM_SUBCORES

        @pl.when(fv < N_FIELDS)
        def _():
            do_field(fv)

    plsc.subcore_barrier()

    pltpu.sync_copy(cnts_hbm.at[c], cnt26_v)

    def counts_of(f):
        row = cnt26_v[f, pl.ds(0, 16)]
        return [row[t] for t in range(NQ)]

    def offs_of(cnts):
        offs = [jnp.int32(0)]
        for t in range(NQ - 1):
            offs.append(offs[t] + _ncf(cnts[t]) * CHK)
        return offs

    def list_xfer(f, buf, fire):
        cnts = counts_of(f)
        offs = offs_of(cnts)
        lb = lbuf_a if buf == 0 else lbuf_b
        for t in range(NQ):
            def cc_body(cc, _):
                cp = pltpu.make_async_copy(
                    lists_hbm.at[c, f, t, pl.ds(cc * CHK, CHK)],
                    lb.at[pl.ds(offs[t] + cc * CHK, CHK)], lsem)
                if fire:
                    cp.start()
                else:
                    cp.wait()
                return 0
            lax.fori_loop(0, _ncf(cnts[t]), cc_body, 0)

    def slice_cp(f, d, t):
        return pltpu.make_async_copy(
            t_hbm.at[f, d, pl.ds(Q_STARTS[t], Q_LENS[t])],
            SB_OF[t], SEM_OF[t])

    zeros = jnp.zeros((16,), jnp.float32)

    def dim_body(dl, _):
        d = w * D_PER_W + dl

        @plsc.parallel_loop(0, (BATCH + 16) // 16, unroll=8)
        def zero_body(j):
            acc_v[pl.ds(j * 16, 16)] = zeros

        list_xfer(0, 0, True)
        slice_cp(0, d, 0).start()
        slice_cp(0, d, 1).start()
        slice_cp(0, d, 2).start()
        slice_cp(0, d, 5).start()
        list_xfer(0, 0, False)

        def pair_body(fp, _):
            for fe in range(2):
                f = fp * 2 + fe
                lb = lbuf_a if fe == 0 else lbuf_b

                @pl.when(f < N_FIELDS - 1)
                def _():
                    list_xfer(f + 1, 1 - fe, True)

                cnts = counts_of(f)
                offs = offs_of(cnts)
                for t in range(NQ):
                    sb = SB_OF[t]
                    slice_cp(f, d, t).wait()
                    base = offs[t]
                    iters = (cnts[t] + 15) >> 4

                    @plsc.parallel_loop(0, iters, unroll=8)
                    def gather_body(j):
                        e = plsc.bitcast(lb[pl.ds(base + j * 16, 16)],
                                         jnp.int32)
                        x = e & jnp.int32(0xFFFF)
                        p = lax.shift_right_logical(e, 16)
                        g = plsc.load_gather(sb, [x])
                        plsc.addupdate_scatter(acc_v, [p], g)

                    df, t2s = NEXT_OF[t]
                    fnext = f + df

                    @pl.when(fnext < N_FIELDS)
                    def _():
                        slice_cp(fnext, d, t2s).start()

                @pl.when(f < N_FIELDS - 1)
                def _():
                    list_xfer(f + 1, 1 - fe, False)

            return 0

        lax.fori_loop(0, N_FIELDS // 2, pair_body, 0)
        pltpu.sync_copy(acc_v.at[pl.ds(0, BATCH)], out_hbm.at[d])
        return 0

    lax.fori_loop(0, D_PER_W, dim_body, 0)


@jax.jit
def _embed_sum(t_tr, feats_flat):
    mesh = plsc.VectorSubcoreMesh(core_axis_name="c", subcore_axis_name="s")
    kfn = pl.kernel(
        _sc_body,
        out_type=(
            jax.ShapeDtypeStruct((EMBED_DIM, BATCH), jnp.float32),
            jax.ShapeDtypeStruct((NUM_CORES, N_FIELDS, NQ, SLOT),
                                 jnp.float32),
            jax.ShapeDtypeStruct((NUM_CORES, N_FIELDS, 16), jnp.int32),
        ),
        mesh=mesh,
        scratch_types=[
            pltpu.VMEM((SBUF,), jnp.float32),
            pltpu.VMEM((SBUF,), jnp.float32),
            pltpu.VMEM((SBUF,), jnp.float32),
            pltpu.VMEM((SBUFD,), jnp.float32),
            pltpu.VMEM((LBUF,), jnp.float32),
            pltpu.VMEM((LBUF,), jnp.float32),
            pltpu.VMEM((BATCH + 16,), jnp.float32),
            pltpu.VMEM((IDXC,), jnp.int32),
            pltpu.VMEM((16,), jnp.int32),
            pltpu.VMEM((N_FIELDS, 16), jnp.int32),
            pltpu.SemaphoreType.DMA,
            pltpu.SemaphoreType.DMA,
            pltpu.SemaphoreType.DMA,
            pltpu.SemaphoreType.DMA,
            pltpu.SemaphoreType.DMA,
        ],
        compiler_params=pltpu.CompilerParams(use_tc_tiling_on_sc=True,
                                             needs_layout_passes=False),
    )
    return kfn(t_tr, feats_flat)


def kernel(features, tables):
    t_tr = tables.transpose(0, 2, 1)
    feats_flat = features.reshape(N_FIELDS * BATCH)
    out_t, _, _ = _embed_sum(t_tr, feats_flat)
    return out_t.T


# speedup vs baseline: 1.1250x; 1.0103x over previous
import jax
import jax.numpy as jnp
from jax import lax
from jax.experimental import pallas as pl
from jax.experimental.pallas import tpu as pltpu
from jax.experimental.pallas import tpu_sc as plsc

N_FIELDS = 26
BATCH = 16384
VOCAB = 100000
EMBED_DIM = 64

NUM_CORES = 2
NUM_SUBCORES = 16
NUM_WORKERS = NUM_CORES * NUM_SUBCORES
D_PER_W = EMBED_DIM // NUM_WORKERS

NQ = 6
Q_STARTS = (0, 16640, 33280, 49920, 66560, 83200)
Q_LENS = (16768, 16768, 16768, 16768, 16768, 16800)
SBUF = 16768
SBUFD = 16800
SLOT = 16384
LBUF = 20480
CHK = 512
IDXC = 2048
DUMP = BATCH


def _ncf(cnt):
    return (((cnt + 15) >> 4 << 4) + CHK - 1) // CHK


def _sc_body(t_hbm, feats_hbm, out_hbm, lists_hbm, cnts_hbm,
             sbuf_a, sbuf_b, sbuf_c, sbuf_d, lbuf_a, lbuf_b,
             acc_v, idx_v, cnt_v, cnt26_v, sem0, sem1, sem2, sem3, lsem):
    c = lax.axis_index("c")
    s = lax.axis_index("s")
    w = c * NUM_SUBCORES + s

    iota16 = lax.iota(jnp.int32, 16)
    build = [sbuf_a, sbuf_b, sbuf_c, sbuf_d, lbuf_a, lbuf_b]
    SB_OF = (sbuf_a, sbuf_b, sbuf_c, sbuf_a, sbuf_b, sbuf_d)
    SEM_OF = (sem0, sem1, sem2, sem0, sem1, sem3)
    NEXT_OF = ((0, 3), (0, 4), (1, 2), (1, 0), (1, 1), (1, 5))

    def do_field(fv):
        def chunk_body(k, tails):
            pltpu.sync_copy(
                feats_hbm.at[pl.ds(fv * BATCH + k * IDXC, IDXC)], idx_v)

            def grp_body(j, tails):
                iv = idx_v[pl.ds(j * 16, 16)]
                ph = ((k * IDXC + j * 16) + iota16) << 16
                new_tails = []
                for t in range(NQ):
                    lo = Q_STARTS[t]
                    hi = Q_STARTS[t + 1] if t + 1 < NQ else VOCAB
                    m = (iv >= lo) & (iv < hi) if t > 0 else (iv < hi)
                    plsc.store_compressed(
                        build[t].at[pl.ds(tails[t], 16)],
                        plsc.bitcast((iv - lo) | ph, jnp.float32), mask=m)
                    pc = jnp.max(plsc.all_reduce_population_count(m))
                    new_tails.append(tails[t] + pc)
                return tuple(new_tails)

            return lax.fori_loop(0, IDXC // 16, grp_body, tails)

        z = jnp.int32(0)
        tails = lax.fori_loop(0, BATCH // IDXC, chunk_body,
                              (z, z, z, z, z, z))

        pad = plsc.bitcast((DUMP + iota16) << 16, jnp.float32)
        cvec = jnp.zeros((16,), jnp.int32)
        for t in range(NQ):
            build[t][pl.ds(tails[t], 16)] = pad
            cvec = jnp.where(iota16 == t, tails[t], cvec)
        cnt_v[pl.ds(0, 16)] = cvec
        pltpu.sync_copy(cnt_v, cnts_hbm.at[c, fv])

        for t in range(NQ):
            def cc_body(cc, _):
                pltpu.sync_copy(
                    build[t].at[pl.ds(cc * CHK, CHK)],
                    lists_hbm.at[c, fv, t, pl.ds(cc * CHK, CHK)])
                return 0
            lax.fori_loop(0, _ncf(tails[t]), cc_body, 0)

    for rep in range(2):
        fv = s + rep * NUM_SUBCORES

        @pl.when(fv < N_FIELDS)
        def _():
            do_field(fv)

    plsc.subcore_barrier()

    pltpu.sync_copy(cnts_hbm.at[c], cnt26_v)

    def counts_of(f):
        row = cnt26_v[f, pl.ds(0, 16)]
        return [row[t] for t in range(NQ)]

    def offs_of(cnts):
        offs = [jnp.int32(0)]
        for t in range(NQ - 1):
            offs.append(offs[t] + _ncf(cnts[t]) * CHK)
        return offs

    def list_xfer(f, buf, fire):
        cnts = counts_of(f)
        offs = offs_of(cnts)
        lb = lbuf_a if buf == 0 else lbuf_b
        for t in range(NQ):
            def cc_body(cc, _):
                cp = pltpu.make_async_copy(
                    lists_hbm.at[c, f, t, pl.ds(cc * CHK, CHK)],
                    lb.at[pl.ds(offs[t] + cc * CHK, CHK)], lsem)
                if fire:
                    cp.start()
                else:
                    cp.wait()
                return 0
            lax.fori_loop(0, _ncf(cnts[t]), cc_body, 0)

    def slice_cp(f, d, t):
        return pltpu.make_async_copy(
            t_hbm.at[f, d, pl.ds(Q_STARTS[t], Q_LENS[t])],
            SB_OF[t], SEM_OF[t])

    zeros = jnp.zeros((16,), jnp.float32)

    def dim_body(dl, _):
        d = w * D_PER_W + dl

        @plsc.parallel_loop(0, (BATCH + 16) // 16, unroll=8)
        def zero_body(j):
            acc_v[pl.ds(j * 16, 16)] = zeros

        list_xfer(0, 0, True)
        slice_cp(0, d, 0).start()
        slice_cp(0, d, 1).start()
        slice_cp(0, d, 2).start()
        slice_cp(0, d, 5).start()
        list_xfer(0, 0, False)

        def pair_body(fp, _):
            for fe in range(2):
                f = fp * 2 + fe
                lb = lbuf_a if fe == 0 else lbuf_b

                @pl.when(f < N_FIELDS - 1)
                def _():
                    list_xfer(f + 1, 1 - fe, True)

                cnts = counts_of(f)
                offs = offs_of(cnts)
                for t in range(NQ):
                    sb = SB_OF[t]
                    slice_cp(f, d, t).wait()
                    base = offs[t]
                    iters = (cnts[t] + 15) >> 4

                    @plsc.parallel_loop(0, iters, unroll=16)
                    def gather_body(j):
                        e = plsc.bitcast(lb[pl.ds(base + j * 16, 16)],
                                         jnp.int32)
                        x = e & jnp.int32(0xFFFF)
                        p = lax.shift_right_logical(e, 16)
                        g = plsc.load_gather(sb, [x])
                        plsc.addupdate_scatter(acc_v, [p], g)

                    df, t2s = NEXT_OF[t]
                    fnext = f + df

                    @pl.when(fnext < N_FIELDS)
                    def _():
                        slice_cp(fnext, d, t2s).start()

                @pl.when(f < N_FIELDS - 1)
                def _():
                    list_xfer(f + 1, 1 - fe, False)

            return 0

        lax.fori_loop(0, N_FIELDS // 2, pair_body, 0)
        pltpu.sync_copy(acc_v.at[pl.ds(0, BATCH)], out_hbm.at[d])
        return 0

    lax.fori_loop(0, D_PER_W, dim_body, 0)


@jax.jit
def _embed_sum(t_tr, feats_flat):
    mesh = plsc.VectorSubcoreMesh(core_axis_name="c", subcore_axis_name="s")
    kfn = pl.kernel(
        _sc_body,
        out_type=(
            jax.ShapeDtypeStruct((EMBED_DIM, BATCH), jnp.float32),
            jax.ShapeDtypeStruct((NUM_CORES, N_FIELDS, NQ, SLOT),
                                 jnp.float32),
            jax.ShapeDtypeStruct((NUM_CORES, N_FIELDS, 16), jnp.int32),
        ),
        mesh=mesh,
        scratch_types=[
            pltpu.VMEM((SBUF,), jnp.float32),
            pltpu.VMEM((SBUF,), jnp.float32),
            pltpu.VMEM((SBUF,), jnp.float32),
            pltpu.VMEM((SBUFD,), jnp.float32),
            pltpu.VMEM((LBUF,), jnp.float32),
            pltpu.VMEM((LBUF,), jnp.float32),
            pltpu.VMEM((BATCH + 16,), jnp.float32),
            pltpu.VMEM((IDXC,), jnp.int32),
            pltpu.VMEM((16,), jnp.int32),
            pltpu.VMEM((N_FIELDS, 16), jnp.int32),
            pltpu.SemaphoreType.DMA,
            pltpu.SemaphoreType.DMA,
            pltpu.SemaphoreType.DMA,
            pltpu.SemaphoreType.DMA,
            pltpu.SemaphoreType.DMA,
        ],
        compiler_params=pltpu.CompilerParams(use_tc_tiling_on_sc=True,
                                             needs_layout_passes=False),
    )
    return kfn(t_tr, feats_flat)


def kernel(features, tables):
    t_tr = tables.transpose(0, 2, 1)
    feats_flat = features.reshape(N_FIELDS * BATCH)
    out_t, _, _ = _embed_sum(t_tr, feats_flat)
    return out_t.T
